# Initial kernel scaffold; baseline (speedup 1.0000x reference)
import jax
import jax.numpy as jnp
from jax.experimental import pallas as pl


def kernel(heatmap):
    raise NotImplementedError("write your pallas kernel here")


# speedup vs baseline: 23.2787x; 23.2787x over previous
import functools

import jax
import jax.numpy as jnp
from jax import lax
from jax.experimental import pallas as pl
from jax.experimental.pallas import tpu as pltpu
import jax.experimental.pallas.tpu_sc as plsc

NC = 2
NS = 16
L = 16
NBATCH = 16
H = W = 512
N = H * W
HALF = N // 2
NWIN = 8
WIN = HALF // NWIN
VPW = WIN // L
K = 256
KEEP = 64
CAP = 288
NB12 = 2048
import numpy as _np
R2 = float(_np.float32(0.05) * _np.float32(0.05))


def _iota():
    return lax.iota(jnp.int32, L)


def _sload(ref, idx):
    return ref[pl.ds(idx, L)][0]


def _monotone(vf):
    b = lax.bitcast_convert_type(vf, jnp.int32)
    return jnp.where(b < 0, b ^ jnp.int32(0x7FFFFFFF), b)


def _clear_hist(hist):
    zeros = jnp.zeros((L,), jnp.int32)
    def body(j, _):
        hist[pl.ds(j * L, L)] = zeros
        return 0
    lax.fori_loop(0, (L * NB12) // L, body, 0, unroll=8)


def _hist_pass(x_hbm, win, hist, base, digit_fn):
    iota = _iota()
    ones = jnp.ones((L,), jnp.int32)

    def wbody(w, _):
        pltpu.sync_copy(x_hbm.at[pl.ds(base + w * WIN, WIN)], win)
        def vbody(i, _):
            v = win[pl.ds(i * L, L)]
            sv = _monotone(v)
            digit, mask = digit_fn(sv)
            plsc.addupdate_scatter(hist, [iota * NB12 + digit], ones, mask=mask)
            return 0
        lax.fori_loop(0, VPW, vbody, 0, unroll=8)
        return 0
    lax.fori_loop(0, NWIN, wbody, 0, unroll=False)


def _reduce_hist(hist, tot):
    def body(j, _):
        acc = jnp.zeros((L,), jnp.int32)
        for l in range(L):
            acc = acc + hist[pl.ds(l * NB12 + j * L, L)]
        tot[pl.ds(j * L, L)] = acc
        return 0
    lax.fori_loop(0, NB12 // L, body, 0, unroll=False)


def _merge_tot(tot, ptot, sh_tot, s):
    pltpu.sync_copy(tot, sh_tot.at[pl.ds(s * NB12, NB12)])
    plsc.subcore_barrier()
    pltpu.sync_copy(sh_tot.at[pl.ds((s ^ 1) * NB12, NB12)], ptot)
    def body(j, _):
        tot[pl.ds(j * L, L)] = tot[pl.ds(j * L, L)] + ptot[pl.ds(j * L, L)]
        return 0
    lax.fori_loop(0, NB12 // L, body, 0, unroll=False)
    plsc.subcore_barrier()


def _find_bin(tot, need, nbins):
    iota = _iota()

    def body(jr, carry):
        csum, bsel, done = carry
        basebin = nbins - L * (jr + 1)
        v = tot[pl.ds(basebin, L)]
        rv = lax.rev(v, (0,))
        c = plsc.cumsum(rv) + csum
        crossed = c >= need
        bins_v = jnp.full((L,), basebin + (L - 1), jnp.int32) - iota
        bcand = jnp.max(jnp.where(crossed, bins_v, jnp.int32(-1)))
        anyc = bcand >= 0
        bsel = jnp.where(jnp.logical_and(jnp.logical_not(done), anyc),
                         bcand, bsel)
        done = jnp.logical_or(done, anyc)
        csum = csum + jnp.sum(rv)
        return csum, bsel, done

    _, bsel, _ = lax.fori_loop(0, nbins // L, body,
                               (jnp.int32(0), jnp.int32(0), False),
                               unroll=False)

    def body2(j, acc):
        v = tot[pl.ds(j * L, L)]
        binidx = jnp.full((L,), j * L, jnp.int32) + iota
        return acc + jnp.sum(jnp.where(binidx > bsel, v, 0))
    above = lax.fori_loop(0, nbins // L, body2, jnp.int32(0), unroll=False)
    return bsel, above


def _collect(x_hbm, win, base, thr, stu, sti, equ, eqi):
    iota = _iota()

    def wbody(w, carry):
        ns, ne = carry
        pltpu.sync_copy(x_hbm.at[pl.ds(base + w * WIN, WIN)], win)

        def vbody(i, carry2):
            ns, ne = carry2
            v = win[pl.ds(i * L, L)]
            sv = _monotone(v)
            gidx = jnp.full((L,), w * WIN + i * L, jnp.int32) + iota
            sm = sv > thr
            em = jnp.logical_and(sv == thr, ne < jnp.int32(256))
            smi = sm.astype(jnp.int32)
            emi = em.astype(jnp.int32)
            spos = ns + plsc.cumsum(smi) - 1
            epos = ne + plsc.cumsum(emi) - 1
            sm = jnp.logical_and(sm, spos < jnp.int32(CAP - L))
            em = jnp.logical_and(em, epos < jnp.int32(CAP - L))
            spos = jnp.clip(spos, 0, CAP - L - 1)
            epos = jnp.clip(epos, 0, CAP - L - 1)
            plsc.store_scatter(stu, [spos], sv, mask=sm)
            plsc.store_scatter(sti, [spos], gidx, mask=sm)
            plsc.store_scatter(equ, [epos], sv, mask=em)
            plsc.store_scatter(eqi, [epos], gidx, mask=em)
            return ns + jnp.sum(smi), ne + jnp.sum(emi)

        return lax.fori_loop(0, VPW, vbody, (ns, ne), unroll=8)

    return lax.fori_loop(0, NWIN, wbody, (jnp.int32(0), jnp.int32(0)),
                         unroll=False)


def _sc_body(x_hbm, out_hbm, win, hist, tot, ptot,
             stu, sti, equ, eqi, pstu, psti, pequ, peqi,
             cntv, sx, sy, kxa, kya, outb,
             sh_tot, sh_cand, sh_cnt):
    c = lax.axis_index("c")
    s = lax.axis_index("s")
    batch = c * (NBATCH // NC) + s // 2
    half = s % 2
    base = batch * N + half * HALF
    iota = _iota()

    _clear_hist(hist)
    _hist_pass(x_hbm, win, hist, base,
               lambda sv: ((sv >> 21) + 1024, None))
    _reduce_hist(hist, tot)
    _merge_tot(tot, ptot, sh_tot, s)
    b1, above1 = _find_bin(tot, jnp.int32(K), NB12)
    x1 = b1 - 1024
    r1 = jnp.int32(K) - above1

    _clear_hist(hist)
    _hist_pass(x_hbm, win, hist, base,
               lambda sv: ((sv >> 10) & 0x7FF, (sv >> 21) == x1))
    _reduce_hist(hist, tot)
    _merge_tot(tot, ptot, sh_tot, s)
    b2, above2 = _find_bin(tot, r1, NB12)
    p21 = (x1 << 11) | b2
    r2n = r1 - above2

    _clear_hist(hist)
    _hist_pass(x_hbm, win, hist, base,
               lambda sv: (sv & 0x3FF, (sv >> 10) == p21))
    _reduce_hist(hist, tot)
    _merge_tot(tot, ptot, sh_tot, s)
    b3, _ = _find_bin(tot, r2n, NB12 // 2)
    thr = (x1 << 21) | (b2 << 10) | b3

    ns, ne = _collect(x_hbm, win, base, thr, stu, sti, equ, eqi)

    cntv[pl.ds(0, L)] = jnp.where(iota == 0, ns,
                                  jnp.where(iota == 1, ne, jnp.int32(0)))
    pltpu.sync_copy(cntv, sh_cnt.at[pl.ds(s * L, L)])
    pltpu.sync_copy(stu, sh_cand.at[pl.ds((s * 4 + 0) * CAP, CAP)])
    pltpu.sync_copy(sti, sh_cand.at[pl.ds((s * 4 + 1) * CAP, CAP)])
    pltpu.sync_copy(equ, sh_cand.at[pl.ds((s * 4 + 2) * CAP, CAP)])
    pltpu.sync_copy(eqi, sh_cand.at[pl.ds((s * 4 + 3) * CAP, CAP)])
    plsc.subcore_barrier()
    p = s ^ 1
    pltpu.sync_copy(sh_cand.at[pl.ds((p * 4 + 0) * CAP, CAP)], pstu)
    pltpu.sync_copy(sh_cand.at[pl.ds((p * 4 + 1) * CAP, CAP)], psti)
    pltpu.sync_copy(sh_cand.at[pl.ds((p * 4 + 2) * CAP, CAP)], pequ)
    pltpu.sync_copy(sh_cand.at[pl.ds((p * 4 + 3) * CAP, CAP)], peqi)
    pltpu.sync_copy(sh_cnt.at[pl.ds(p * L, L)], cntv)
    plsc.subcore_barrier()
    cnt16 = cntv[pl.ds(0, L)]
    pns = cnt16[0]
    pne = cnt16[1]
    my_off = half * HALF
    pr_off = (1 - half) * HALF

    @pl.when(half == 0)
    def _rank_and_nms():
        inv511 = jnp.float32(1.0) / jnp.float32(511.0)
        segs = [
            (stu, sti, ns, my_off),
            (pstu, psti, pns, pr_off),
            (equ, eqi, ne, my_off),
            (pequ, peqi, pne, pr_off),
        ]
        segs = [(a, b, jnp.minimum(n, jnp.int32(CAP - L)), o)
                for (a, b, n, o) in segs]

        for (ou, oi, on, ooff) in segs:
            def obody(jv, _):
                bidx = jv * L
                ju = ou[pl.ds(bidx, L)]
                ji = oi[pl.ds(bidx, L)] + ooff
                valid = (jnp.full((L,), bidx, jnp.int32) + iota) < on
                rank = jnp.zeros((L,), jnp.int32)
                for (iu, ii, inn, ioff) in segs:
                    def ibody(k2, rk):
                        ku = _sload(iu, k2)
                        ki = _sload(ii, k2) + ioff
                        gt = ku > ju
                        tie = jnp.logical_and(ku == ju, ki < ji)
                        return rk + jnp.logical_or(gt, tie).astype(jnp.int32)
                    rank = lax.fori_loop(0, inn, ibody, rank, unroll=False)
                keepm = jnp.logical_and(valid, rank < jnp.int32(K))
                rank = jnp.where(keepm, rank, jnp.int32(0))
                xf = (ji & 511).astype(jnp.float32) * inv511
                yf = (ji >> 9).astype(jnp.float32) * inv511
                plsc.store_scatter(sx, [rank], xf, mask=keepm)
                plsc.store_scatter(sy, [rank], yf, mask=keepm)
                return 0
            lax.fori_loop(0, (on + L - 1) // L, obody, 0, unroll=False)

        slots = [jnp.full((L,), g * L, jnp.int32) + iota for g in range(4)]
        zf = jnp.zeros((L,), jnp.float32)

        def nbody(i, carry):
            cnt, kx0, kx1, kx2, kx3, ky0, ky1, ky2, ky3 = carry
            kxs = [kx0, kx1, kx2, kx3]
            kys = [ky0, ky1, ky2, ky3]
            xi = _sload(sx, i)
            yi = _sload(sy, i)
            close = None
            for g in range(4):
                dx = kxs[g] - xi
                dy = kys[g] - yi
                d2 = dx * dx + dy * dy
                cg = jnp.logical_and(d2 < R2, slots[g] < cnt)
                close = cg if close is None else jnp.logical_or(close, cg)
            too_close = jnp.any(close)
            do_add = jnp.logical_and(jnp.logical_not(too_close),
                                     cnt < jnp.int32(KEEP))
            for g in range(4):
                sel = jnp.logical_and(do_add, slots[g] == cnt)
                kxs[g] = jnp.where(sel, xi, kxs[g])
                kys[g] = jnp.where(sel, yi, kys[g])
            cnt = cnt + do_add.astype(jnp.int32)
            return (cnt, kxs[0], kxs[1], kxs[2], kxs[3],
                    kys[0], kys[1], kys[2], kys[3])

        init = (jnp.int32(0), zf, zf, zf, zf, zf, zf, zf, zf)
        res = lax.fori_loop(0, K, nbody, init, unroll=False)
        cnt = res[0]
        for g in range(4):
            kxa[pl.ds(g * L, L)] = res[1 + g]
            kya[pl.ds(g * L, L)] = res[5 + g]
        lastx = _sload(kxa, cnt - 1)
        lasty = _sload(kya, cnt - 1)
        for g in range(4):
            valid = slots[g] < cnt
            xg = jnp.where(valid, res[1 + g], lastx)
            yg = jnp.where(valid, res[5 + g], lasty)
            plsc.store_scatter(outb, [slots[g] * 2], xg)
            plsc.store_scatter(outb, [slots[g] * 2 + 1], yg)
        pltpu.sync_copy(outb, out_hbm.at[pl.ds(batch * KEEP * 2, KEEP * 2)])


@functools.lru_cache(maxsize=1)
def _build():
    mesh = plsc.VectorSubcoreMesh(core_axis_name="c", subcore_axis_name="s",
                                  num_cores=NC, num_subcores=NS)
    return pl.kernel(
        _sc_body,
        out_type=jax.ShapeDtypeStruct((NBATCH * KEEP * 2,), jnp.float32),
        mesh=mesh,
        compiler_params=pltpu.CompilerParams(needs_layout_passes=False),
        scratch_types=[
            pltpu.VMEM((WIN,), jnp.float32),
            pltpu.VMEM((L * NB12,), jnp.int32),
            pltpu.VMEM((NB12,), jnp.int32),
            pltpu.VMEM((NB12,), jnp.int32),
            pltpu.VMEM((CAP,), jnp.int32),
            pltpu.VMEM((CAP,), jnp.int32),
            pltpu.VMEM((CAP,), jnp.int32),
            pltpu.VMEM((CAP,), jnp.int32),
            pltpu.VMEM((CAP,), jnp.int32),
            pltpu.VMEM((CAP,), jnp.int32),
            pltpu.VMEM((CAP,), jnp.int32),
            pltpu.VMEM((CAP,), jnp.int32),
            pltpu.VMEM((L,), jnp.int32),
            pltpu.VMEM((K + L,), jnp.float32),
            pltpu.VMEM((K + L,), jnp.float32),
            pltpu.VMEM((KEEP + L,), jnp.float32),
            pltpu.VMEM((KEEP + L,), jnp.float32),
            pltpu.VMEM((KEEP * 2,), jnp.float32),
            pltpu.VMEM_SHARED((NS * NB12,), jnp.int32),
            pltpu.VMEM_SHARED((NS * 4 * CAP,), jnp.int32),
            pltpu.VMEM_SHARED((NS * L,), jnp.int32),
        ],
    )


def kernel(heatmap):
    flat = heatmap.reshape(-1)
    out = _build()(flat)
    return out.reshape(NBATCH, KEEP, 2)


# speedup vs baseline: 25.2252x; 1.0836x over previous
import functools

import jax
import jax.numpy as jnp
import numpy as _np
from jax import lax
from jax.experimental import pallas as pl
from jax.experimental.pallas import tpu as pltpu
import jax.experimental.pallas.tpu_sc as plsc

NC = 2
NS = 16
L = 16
NBATCH = 16
H = W = 512
N = H * W
HALF = N // 2
NWIN = 8
WIN = HALF // NWIN
VPW = WIN // L
K = 256
KEEP = 64
CAP = 560
NB12 = 2048
FASTCAP = 271
R2 = float(_np.float32(0.05) * _np.float32(0.05))
IMIN = -2**31
IBIG = 2**30


def _iota():
    return lax.iota(jnp.int32, L)


def _sload(ref, idx):
    return ref[pl.ds(idx, L)][0]


def _monotone(vf):
    b = lax.bitcast_convert_type(vf, jnp.int32)
    return jnp.where(b < 0, b ^ jnp.int32(0x7FFFFFFF), b)


def _clear_hist(hist):
    zeros = jnp.zeros((L,), jnp.int32)
    def body(j, _):
        hist[pl.ds(j * L, L)] = zeros
        return 0
    lax.fori_loop(0, (L * NB12) // L, body, 0, unroll=8)


def _windowed(x_hbm, wins, sems, base, process, init):
    def start(w, b):
        pltpu.async_copy(x_hbm.at[pl.ds(base + w * WIN, WIN)], wins[b],
                         sems[b])

    def wait(b):
        pltpu.make_async_copy(x_hbm.at[pl.ds(base, WIN)], wins[b],
                              sems[b]).wait()

    start(0, 0)
    start(1, 1)

    def body(p, carry):
        w = p * 2
        wait(0)
        carry = process(wins[0], w, carry)
        start(jnp.minimum(w + 2, NWIN - 1), 0)
        wait(1)
        carry = process(wins[1], w + 1, carry)
        start(jnp.minimum(w + 3, NWIN - 1), 1)
        return carry

    carry = lax.fori_loop(0, NWIN // 2, body, init, unroll=False)
    wait(0)
    wait(1)
    return carry


def _hist_pass(x_hbm, wins, sems, hist, base, digit_fn):
    iota = _iota()
    ones = jnp.ones((L,), jnp.int32)

    def process(buf, w, carry):
        def vbody(i, _):
            v = buf[pl.ds(i * L, L)]
            sv = _monotone(v)
            digit, mask = digit_fn(sv)
            plsc.addupdate_scatter(hist, [iota * NB12 + digit], ones, mask=mask)
            return 0
        lax.fori_loop(0, VPW, vbody, 0, unroll=8)
        return carry
    _windowed(x_hbm, wins, sems, base, process, jnp.int32(0))


def _reduce_hist(hist, tot):
    def body(j, _):
        acc = jnp.zeros((L,), jnp.int32)
        for l in range(L):
            acc = acc + hist[pl.ds(l * NB12 + j * L, L)]
        tot[pl.ds(j * L, L)] = acc
        return 0
    lax.fori_loop(0, NB12 // L, body, 0, unroll=False)


def _merge_tot(tot, ptot, sh_tot, s):
    pltpu.sync_copy(tot, sh_tot.at[pl.ds(s * NB12, NB12)])
    plsc.subcore_barrier()
    pltpu.sync_copy(sh_tot.at[pl.ds((s ^ 1) * NB12, NB12)], ptot)
    def body(j, _):
        tot[pl.ds(j * L, L)] = tot[pl.ds(j * L, L)] + ptot[pl.ds(j * L, L)]
        return 0
    lax.fori_loop(0, NB12 // L, body, 0, unroll=False)
    plsc.subcore_barrier()


def _find_bin(tot, need, nbins):
    iota = _iota()

    def body(jr, carry):
        csum, bsel, done = carry
        basebin = nbins - L * (jr + 1)
        v = tot[pl.ds(basebin, L)]
        rv = lax.rev(v, (0,))
        cum = plsc.cumsum(rv) + csum
        crossed = cum >= need
        bins_v = jnp.full((L,), basebin + (L - 1), jnp.int32) - iota
        bcand = jnp.max(jnp.where(crossed, bins_v, jnp.int32(-1)))
        anyc = bcand >= 0
        bsel = jnp.where(jnp.logical_and(jnp.logical_not(done), anyc),
                         bcand, bsel)
        done = jnp.logical_or(done, anyc)
        csum = csum + jnp.sum(rv)
        return csum, bsel, done

    _, bsel, _ = lax.fori_loop(0, nbins // L, body,
                               (jnp.int32(0), jnp.int32(0), False),
                               unroll=False)

    def body2(j, acc):
        v = tot[pl.ds(j * L, L)]
        binidx = jnp.full((L,), j * L, jnp.int32) + iota
        return acc + jnp.sum(jnp.where(binidx > bsel, v, 0))
    above = lax.fori_loop(0, nbins // L, body2, jnp.int32(0), unroll=False)
    return bsel, above


def _collect(x_hbm, wins, sems, base, thr, cu, ci):
    iota = _iota()

    def process(buf, w, carry):
        def vbody(i, carry2):
            cnt, ne = carry2
            v = buf[pl.ds(i * L, L)]
            sv = _monotone(v)
            gidx = jnp.full((L,), w * WIN + i * L, jnp.int32) + iota
            sm = sv > thr
            em = jnp.logical_and(sv == thr, ne < jnp.int32(256))
            m = jnp.logical_or(sm, em)
            mi = m.astype(jnp.int32)
            pos = cnt + plsc.cumsum(mi) - 1
            m = jnp.logical_and(m, pos < jnp.int32(CAP - L))
            pos = jnp.clip(pos, 0, CAP - L - 1)
            plsc.store_scatter(cu, [pos], sv, mask=m)
            plsc.store_scatter(ci, [pos], gidx, mask=m)
            return cnt + jnp.sum(mi), ne + jnp.sum(em.astype(jnp.int32))

        return lax.fori_loop(0, VPW, vbody, carry, unroll=8)

    cnt, _ne = _windowed(x_hbm, wins, sems, base, process,
                         (jnp.int32(0), jnp.int32(0)))
    return cnt


def _clear_tail(keys, idxs, n):
    iota = _iota()
    kfill = jnp.full((L,), jnp.int32(IMIN))
    ifill = jnp.full((L,), jnp.int32(IBIG))
    def body(j, _):
        lanes = jnp.full((L,), j * L, jnp.int32) + iota
        m = lanes >= n
        kv = keys[pl.ds(j * L, L)]
        iv = idxs[pl.ds(j * L, L)]
        keys[pl.ds(j * L, L)] = jnp.where(m, kfill, kv)
        idxs[pl.ds(j * L, L)] = jnp.where(m, ifill, iv)
        return 0
    lax.fori_loop(0, CAP // L, body, 0, unroll=4)


def _exchange_max(val, sh_cnt, cntv, scratch, s):
    cntv[pl.ds(0, L)] = jnp.full((L,), val, jnp.int32)
    pltpu.sync_copy(cntv, sh_cnt.at[pl.ds(s * L, L)])
    plsc.subcore_barrier()
    pltpu.sync_copy(sh_cnt, scratch.at[pl.ds(0, NS * L)])
    def body(j, acc):
        return jnp.maximum(acc, jnp.max(scratch[pl.ds(j * L, L)]))
    m = lax.fori_loop(0, NS, body, jnp.int32(IMIN), unroll=False)
    plsc.subcore_barrier()
    return m


def _sc_body(x_hbm, out_hbm, win, win2, hist, tot, ptot,
             cu, ci, pcu, pci,
             cntv, sx, sy, kxa, kya, outb, sem0, sem1,
             sh_tot, sh_cand, sh_cnt):
    c = lax.axis_index("c")
    s = lax.axis_index("s")
    batch = c * (NBATCH // NC) + s // 2
    half = s % 2
    base = batch * N + half * HALF
    iota = _iota()
    wins = (win, win2)
    sems = (sem0, sem1)

    _clear_hist(hist)
    _hist_pass(x_hbm, wins, sems, hist, base,
               lambda sv: ((sv >> 21) + 1024, None))
    _reduce_hist(hist, tot)
    _merge_tot(tot, ptot, sh_tot, s)
    b1, above1 = _find_bin(tot, jnp.int32(K), NB12)
    x1 = b1 - 1024
    r1 = jnp.int32(K) - above1
    c_geq = above1 + _sload(tot, b1)

    slow = _exchange_max(c_geq, sh_cnt, cntv, ptot, s) > jnp.int32(FASTCAP)

    def _refine(_):
        _clear_hist(hist)
        _hist_pass(x_hbm, wins, sems, hist, base,
                   lambda sv: ((sv >> 10) & 0x7FF, (sv >> 21) == x1))
        _reduce_hist(hist, tot)
        _merge_tot(tot, ptot, sh_tot, s)
        b2, above2 = _find_bin(tot, r1, NB12)
        p21 = (x1 << 11) | b2
        r2n = r1 - above2
        _clear_hist(hist)
        _hist_pass(x_hbm, wins, sems, hist, base,
                   lambda sv: (sv & 0x3FF, (sv >> 10) == p21))
        _reduce_hist(hist, tot)
        _merge_tot(tot, ptot, sh_tot, s)
        b3, _ = _find_bin(tot, r2n, NB12 // 2)
        return (x1 << 21) | (b2 << 10) | b3

    def _fast(_):
        return (x1 << 21) - 1

    thr = lax.cond(slow, _refine, _fast, 0)

    cnt = _collect(x_hbm, wins, sems, base, thr, cu, ci)
    _clear_tail(cu, ci, cnt)

    cntv[pl.ds(0, L)] = jnp.where(iota == 0, cnt, jnp.int32(0))
    pltpu.sync_copy(cntv, sh_cnt.at[pl.ds(s * L, L)])
    pltpu.sync_copy(cu, sh_cand.at[pl.ds((s * 2 + 0) * CAP, CAP)])
    pltpu.sync_copy(ci, sh_cand.at[pl.ds((s * 2 + 1) * CAP, CAP)])
    plsc.subcore_barrier()
    p = s ^ 1
    pltpu.sync_copy(sh_cand.at[pl.ds((p * 2 + 0) * CAP, CAP)], pcu)
    pltpu.sync_copy(sh_cand.at[pl.ds((p * 2 + 1) * CAP, CAP)], pci)
    pltpu.sync_copy(sh_cnt.at[pl.ds(p * L, L)], cntv)
    plsc.subcore_barrier()
    pcnt = cntv[pl.ds(0, L)][0]
    my_off = half * HALF
    pr_off = (1 - half) * HALF

    @pl.when(half == 0)
    def _rank_and_nms():
        inv511 = jnp.float32(1.0) / jnp.float32(511.0)
        segs = [
            (cu, ci, jnp.minimum(cnt, jnp.int32(CAP - L)), my_off),
            (pcu, pci, jnp.minimum(pcnt, jnp.int32(CAP - L)), pr_off),
        ]

        for (ou, oi, on, ooff) in segs:
            def obody(jv, _):
                bidx = jv * L
                ju = ou[pl.ds(bidx, L)]
                ji = oi[pl.ds(bidx, L)] + ooff
                valid = (jnp.full((L,), bidx, jnp.int32) + iota) < on
                rank = jnp.zeros((L,), jnp.int32)
                for (iu, ii, inn, ioff) in segs:
                    def ibody(kv, rk):
                        kuv = iu[pl.ds(kv * L, L)]
                        kiv = ii[pl.ds(kv * L, L)] + ioff
                        for r in range(L):
                            ku = kuv[r]
                            ki = kiv[r]
                            gt = ku > ju
                            tie = jnp.logical_and(ku == ju, ki < ji)
                            rk = rk + jnp.logical_or(gt, tie).astype(jnp.int32)
                        return rk
                    rank = lax.fori_loop(0, (inn + L - 1) // L, ibody, rank,
                                         unroll=False)
                keepm = jnp.logical_and(valid, rank < jnp.int32(K))
                rank = jnp.where(keepm, rank, jnp.int32(0))
                xf = (ji & 511).astype(jnp.float32) * inv511
                yf = (ji >> 9).astype(jnp.float32) * inv511
                plsc.store_scatter(sx, [rank], xf, mask=keepm)
                plsc.store_scatter(sy, [rank], yf, mask=keepm)
                return 0
            lax.fori_loop(0, (on + L - 1) // L, obody, 0, unroll=False)

        slots = [jnp.full((L,), g * L, jnp.int32) + iota for g in range(4)]
        zf = jnp.zeros((L,), jnp.float32)

        def nbody(i, carry):
            cnt_, kx0, kx1, kx2, kx3, ky0, ky1, ky2, ky3 = carry
            kxs = [kx0, kx1, kx2, kx3]
            kys = [ky0, ky1, ky2, ky3]
            xi = _sload(sx, i)
            yi = _sload(sy, i)
            close = None
            for g in range(4):
                dx = kxs[g] - xi
                dy = kys[g] - yi
                d2 = dx * dx + dy * dy
                cg = jnp.logical_and(d2 < R2, slots[g] < cnt_)
                close = cg if close is None else jnp.logical_or(close, cg)
            too_close = jnp.any(close)
            do_add = jnp.logical_and(jnp.logical_not(too_close),
                                     cnt_ < jnp.int32(KEEP))
            for g in range(4):
                sel = jnp.logical_and(do_add, slots[g] == cnt_)
                kxs[g] = jnp.where(sel, xi, kxs[g])
                kys[g] = jnp.where(sel, yi, kys[g])
            cnt_ = cnt_ + do_add.astype(jnp.int32)
            return (cnt_, kxs[0], kxs[1], kxs[2], kxs[3],
                    kys[0], kys[1], kys[2], kys[3])

        init = (jnp.int32(0), zf, zf, zf, zf, zf, zf, zf, zf)
        res = lax.fori_loop(0, K, nbody, init, unroll=False)
        kcnt = res[0]
        for g in range(4):
            kxa[pl.ds(g * L, L)] = res[1 + g]
            kya[pl.ds(g * L, L)] = res[5 + g]
        lastx = _sload(kxa, kcnt - 1)
        lasty = _sload(kya, kcnt - 1)
        for g in range(4):
            valid = slots[g] < kcnt
            xg = jnp.where(valid, res[1 + g], lastx)
            yg = jnp.where(valid, res[5 + g], lasty)
            plsc.store_scatter(outb, [slots[g] * 2], xg)
            plsc.store_scatter(outb, [slots[g] * 2 + 1], yg)
        pltpu.sync_copy(outb, out_hbm.at[pl.ds(batch * KEEP * 2, KEEP * 2)])


@functools.lru_cache(maxsize=1)
def _build():
    mesh = plsc.VectorSubcoreMesh(core_axis_name="c", subcore_axis_name="s",
                                  num_cores=NC, num_subcores=NS)
    return pl.kernel(
        _sc_body,
        out_type=jax.ShapeDtypeStruct((NBATCH * KEEP * 2,), jnp.float32),
        mesh=mesh,
        compiler_params=pltpu.CompilerParams(needs_layout_passes=False),
        scratch_types=[
            pltpu.VMEM((WIN,), jnp.float32),
            pltpu.VMEM((WIN,), jnp.float32),
            pltpu.VMEM((L * NB12,), jnp.int32),
            pltpu.VMEM((NB12,), jnp.int32),
            pltpu.VMEM((NB12,), jnp.int32),
            pltpu.VMEM((CAP,), jnp.int32),
            pltpu.VMEM((CAP,), jnp.int32),
            pltpu.VMEM((CAP,), jnp.int32),
            pltpu.VMEM((CAP,), jnp.int32),
            pltpu.VMEM((L,), jnp.int32),
            pltpu.VMEM((K + L,), jnp.float32),
            pltpu.VMEM((K + L,), jnp.float32),
            pltpu.VMEM((KEEP + L,), jnp.float32),
            pltpu.VMEM((KEEP + L,), jnp.float32),
            pltpu.VMEM((KEEP * 2,), jnp.float32),
            pltpu.SemaphoreType.DMA,
            pltpu.SemaphoreType.DMA,
            pltpu.VMEM_SHARED((NS * NB12,), jnp.int32),
            pltpu.VMEM_SHARED((NS * 2 * CAP,), jnp.int32),
            pltpu.VMEM_SHARED((NS * L,), jnp.int32),
        ],
    )


def kernel(heatmap):
    flat = heatmap.reshape(-1)
    out = _build()(flat)
    return out.reshape(NBATCH, KEEP, 2)
